# Initial kernel scaffold; baseline (speedup 1.0000x reference)
import jax
import jax.numpy as jnp
from jax.experimental import pallas as pl


def kernel(x, edge_index, W1, as1, ad1, b1, W2, as2, ad2, b2, W3, as3, ad3, b3, W4, as4, ad4, b4, Wfc, bfc):
    raise NotImplementedError("write your pallas kernel here")


# speedup vs baseline: 24.7890x; 24.7890x over previous
import functools

import jax
import jax.numpy as jnp
from jax import lax
from jax.experimental import pallas as pl
from jax.experimental.pallas import tpu as pltpu
from jax.experimental.pallas import tpu_sc as plsc

N = 10000
NE = 330000
NC, NS = 2, 16
NW = NC * NS
CHUNK = 128
EW = 10368
EP = EW * NW
NPAD = 10240
RT = NPAD // NS
PAD_DST = 10000
BR = 1000
G = N // BR

_f32 = jnp.float32


def _hext(h, a_s, a_d):
    es = jnp.sum(h * a_s, axis=1, keepdims=True)
    ed = jnp.sum(h * a_d, axis=1, keepdims=True)
    hx = jnp.concatenate(
        [h, es, ed, jnp.zeros((BR, 14), _f32)], axis=1)
    return hx, jnp.sum(h * a_d, axis=1).reshape(1, 1, BR)


def _tc_first_body(x_ref, w_ref, as_ref, ad_ref, h_ref, ed_ref):
    h = jnp.dot(x_ref[...], w_ref[...], preferred_element_type=_f32)
    h_ref[...], ed_ref[...] = _hext(h, as_ref[...], ad_ref[...])


def _norm_x(acc_ref, b_ref, din):
    s = acc_ref[0] + acc_ref[1]
    den = s[:, din:din + 1]
    return jnp.maximum(s[:, :din] / den + b_ref[...], 0.0)


def _tc_mid_body(din, acc_ref, b_ref, w_ref, as_ref, ad_ref, h_ref, ed_ref):
    x = _norm_x(acc_ref, b_ref, din)
    h = jnp.dot(x, w_ref[...], preferred_element_type=_f32)
    h_ref[...], ed_ref[...] = _hext(h, as_ref[...], ad_ref[...])


def _tc_final_body(acc_ref, b_ref, wfc_ref, bfc_ref, out_ref):
    x = _norm_x(acc_ref, b_ref, 128)
    out_ref[...] = (jnp.dot(x, wfc_ref[...], preferred_element_type=_f32)
                    + bfc_ref[...])


def _whole(shape):
    return pl.BlockSpec(shape, lambda i: (0,) * len(shape))


def _tc_first(x, w, a_s, a_d):
    din, dout = w.shape
    return pl.pallas_call(
        _tc_first_body,
        grid=(G,),
        in_specs=[
            pl.BlockSpec((BR, din), lambda i: (i, 0)),
            _whole((din, dout)), _whole((1, dout)), _whole((1, dout)),
        ],
        out_specs=[
            pl.BlockSpec((BR, dout + 16), lambda i: (i, 0)),
            pl.BlockSpec((1, 1, BR), lambda i: (i, 0, 0)),
        ],
        out_shape=[
            jax.ShapeDtypeStruct((N, dout + 16), _f32),
            jax.ShapeDtypeStruct((G, 1, BR), _f32),
        ],
    )(x, w, a_s, a_d)


def _tc_mid(accp, b, w, a_s, a_d):
    din, dout = w.shape
    dc = din + 16
    return pl.pallas_call(
        functools.partial(_tc_mid_body, din),
        grid=(G,),
        in_specs=[
            pl.BlockSpec((NC, BR, dc), lambda i: (0, i, 0)),
            _whole((1, din)), _whole((din, dout)),
            _whole((1, dout)), _whole((1, dout)),
        ],
        out_specs=[
            pl.BlockSpec((BR, dout + 16), lambda i: (i, 0)),
            pl.BlockSpec((1, 1, BR), lambda i: (i, 0, 0)),
        ],
        out_shape=[
            jax.ShapeDtypeStruct((N, dout + 16), _f32),
            jax.ShapeDtypeStruct((G, 1, BR), _f32),
        ],
    )(accp, b, w, a_s, a_d)


def _tc_final(accp, b, wfc, bfc):
    dc = 128 + 16
    return pl.pallas_call(
        _tc_final_body,
        grid=(G,),
        in_specs=[
            pl.BlockSpec((NC, BR, dc), lambda i: (0, i, 0)),
            _whole((1, 128)), _whole((128, 128)), _whole((1, 128)),
        ],
        out_specs=pl.BlockSpec((BR, 128), lambda i: (i, 0)),
        out_shape=jax.ShapeDtypeStruct((N, 128), _f32),
    )(accp, b, wfc, bfc)


@functools.lru_cache(maxsize=None)
def _sc_edge(dout):
    dc = dout + 16
    cg = dout // 16
    mesh = plsc.VectorSubcoreMesh(core_axis_name="c", subcore_axis_name="s",
                                  num_cores=NC, num_subcores=NS)

    @functools.partial(
        pl.kernel,
        out_type=jax.ShapeDtypeStruct((NC, NPAD, dc), _f32),
        mesh=mesh,
        compiler_params=pltpu.CompilerParams(needs_layout_passes=False,
                                             use_tc_tiling_on_sc=False),
        scratch_types=[
            pltpu.VMEM((NPAD,), _f32),
            pltpu.VMEM((CHUNK,), jnp.int32),
            pltpu.VMEM((CHUNK,), jnp.int32),
            pltpu.VMEM((CHUNK,), _f32),
            pltpu.VMEM((CHUNK, dc), _f32),
            pltpu.VMEM_SHARED((NPAD, dc), _f32),
            pltpu.SemaphoreType.DMA,
        ],
    )
    def sc_fn(ed_hbm, src_hbm, dst_hbm, h_hbm, out_hbm,
              ed_v, sidx, didx, wv, rows, acc, sem):
        cid = lax.axis_index("c")
        sid = lax.axis_index("s")
        wid = cid * NS + sid

        def _zero_row(r, _):
            for g in range(dc // 16):
                rows[r, pl.ds(g * 16, 16)] = jnp.zeros((16,), _f32)
            return 0
        lax.fori_loop(0, CHUNK, _zero_row, 0)
        for k in range(RT // CHUNK):
            pltpu.sync_copy(rows, acc.at[pl.ds(sid * RT + k * CHUNK, CHUNK)])
        plsc.subcore_barrier()

        pltpu.sync_copy(ed_hbm, ed_v)

        def _chunk(j, _):
            base = wid * EW + j * CHUNK
            pltpu.sync_copy(src_hbm.at[pl.ds(base, CHUNK)], sidx)
            pltpu.sync_copy(dst_hbm.at[pl.ds(base, CHUNK)], didx)
            pltpu.async_copy(h_hbm.at[sidx], rows, sem).wait()

            def _wgrp(g, _):
                lanes = lax.iota(jnp.int32, 16)
                col_es = jnp.full((16,), dout, jnp.int32)
                ridx = lanes + g * 16
                e = (plsc.load_gather(rows, [ridx, col_es])
                     + plsc.load_gather(ed_v, [didx[pl.ds(g * 16, 16)]]))
                wv[pl.ds(g * 16, 16)] = jnp.exp(jnp.maximum(e, 0.2 * e))
                return 0
            lax.fori_loop(0, CHUNK // 16, _wgrp, 0)

            def _edge(i, _):
                wbc = plsc.load_gather(wv, [jnp.full((16,), i, jnp.int32)])
                for g in range(cg):
                    rows[i, pl.ds(g * 16, 16)] = (
                        rows[i, pl.ds(g * 16, 16)] * wbc)
                rows[i, pl.ds(dout, 16)] = wbc
                return 0
            lax.fori_loop(0, CHUNK, _edge, 0)

            pltpu.sync_copy(rows, acc.at[didx], add=True)
            return 0
        lax.fori_loop(0, EW // CHUNK, _chunk, 0)

        plsc.subcore_barrier()
        pltpu.sync_copy(acc.at[pl.ds(sid * RT, RT)],
                        out_hbm.at[cid, pl.ds(sid * RT, RT)])

    return sc_fn


def _pad_nodes(v3):
    v = v3.reshape(N)
    return jnp.concatenate([v, jnp.zeros((NPAD - N,), _f32)])


def kernel(x, edge_index, W1, as1, ad1, b1, W2, as2, ad2, b2,
           W3, as3, ad3, b3, W4, as4, ad4, b4, Wfc, bfc):
    loops = jnp.arange(N, dtype=jnp.int32)
    src = jnp.concatenate(
        [edge_index[0], loops, jnp.zeros((EP - NE,), jnp.int32)])
    dst = jnp.concatenate(
        [edge_index[1], loops,
         jnp.full((EP - NE,), PAD_DST, jnp.int32)])

    r2 = lambda a: a.reshape(1, -1)
    h, ed3 = _tc_first(x, W1, r2(as1), r2(ad1))
    accp = _sc_edge(W1.shape[1])(_pad_nodes(ed3), src, dst, h)
    for (W, a_s, a_d, bprev) in ((W2, as2, ad2, b1), (W3, as3, ad3, b2),
                                 (W4, as4, ad4, b3)):
        h, ed3 = _tc_mid(accp, r2(bprev), W, r2(a_s), r2(a_d))
        accp = _sc_edge(W.shape[1])(_pad_nodes(ed3), src, dst, h)
    return _tc_final(accp, r2(b4), Wfc, r2(bfc))


# speedup vs baseline: 33.5477x; 1.3533x over previous
import functools

import jax
import jax.numpy as jnp
from jax import lax
from jax.experimental import pallas as pl
from jax.experimental.pallas import tpu as pltpu
from jax.experimental.pallas import tpu_sc as plsc

N = 10000
NE = 330000
NC, NS = 2, 16
NW = NC * NS
CHUNK = 128
EW = 10368
EP = EW * NW
NPAD = 10240
RT = NPAD // NS
PAD_DST = 10000
BR = 1000
G = N // BR

_f32 = jnp.float32


def _hext(h, a_s, a_d):
    es = jnp.sum(h * a_s, axis=1, keepdims=True)
    ed = jnp.sum(h * a_d, axis=1, keepdims=True)
    hx = jnp.concatenate(
        [h, es, ed, jnp.zeros((BR, 14), _f32)], axis=1)
    return hx, jnp.sum(h * a_d, axis=1).reshape(1, 1, BR)


def _tc_first_body(x_ref, w_ref, as_ref, ad_ref, h_ref, ed_ref):
    h = jnp.dot(x_ref[...], w_ref[...], preferred_element_type=_f32)
    h_ref[...], ed_ref[...] = _hext(h, as_ref[...], ad_ref[...])


def _norm_x(acc_ref, b_ref, din):
    s = acc_ref[0] + acc_ref[1]
    den = s[:, din:din + 1]
    return jnp.maximum(s[:, :din] / den + b_ref[...], 0.0)


def _tc_mid_body(din, acc_ref, b_ref, w_ref, as_ref, ad_ref, h_ref, ed_ref):
    x = _norm_x(acc_ref, b_ref, din)
    h = jnp.dot(x, w_ref[...], preferred_element_type=_f32)
    h_ref[...], ed_ref[...] = _hext(h, as_ref[...], ad_ref[...])


def _tc_final_body(acc_ref, b_ref, wfc_ref, bfc_ref, out_ref):
    x = _norm_x(acc_ref, b_ref, 128)
    out_ref[...] = (jnp.dot(x, wfc_ref[...], preferred_element_type=_f32)
                    + bfc_ref[...])


def _whole(shape):
    return pl.BlockSpec(shape, lambda i: (0,) * len(shape))


def _tc_first(x, w, a_s, a_d):
    din, dout = w.shape
    return pl.pallas_call(
        _tc_first_body,
        grid=(G,),
        in_specs=[
            pl.BlockSpec((BR, din), lambda i: (i, 0)),
            _whole((din, dout)), _whole((1, dout)), _whole((1, dout)),
        ],
        out_specs=[
            pl.BlockSpec((BR, dout + 16), lambda i: (i, 0)),
            pl.BlockSpec((1, 1, BR), lambda i: (i, 0, 0)),
        ],
        out_shape=[
            jax.ShapeDtypeStruct((N, dout + 16), _f32),
            jax.ShapeDtypeStruct((G, 1, BR), _f32),
        ],
    )(x, w, a_s, a_d)


def _tc_mid(accp, b, w, a_s, a_d):
    din, dout = w.shape
    dc = din + 16
    return pl.pallas_call(
        functools.partial(_tc_mid_body, din),
        grid=(G,),
        in_specs=[
            pl.BlockSpec((NC, BR, dc), lambda i: (0, i, 0)),
            _whole((1, din)), _whole((din, dout)),
            _whole((1, dout)), _whole((1, dout)),
        ],
        out_specs=[
            pl.BlockSpec((BR, dout + 16), lambda i: (i, 0)),
            pl.BlockSpec((1, 1, BR), lambda i: (i, 0, 0)),
        ],
        out_shape=[
            jax.ShapeDtypeStruct((N, dout + 16), _f32),
            jax.ShapeDtypeStruct((G, 1, BR), _f32),
        ],
    )(accp, b, w, a_s, a_d)


def _tc_final(accp, b, wfc, bfc):
    dc = 128 + 16
    return pl.pallas_call(
        _tc_final_body,
        grid=(G,),
        in_specs=[
            pl.BlockSpec((NC, BR, dc), lambda i: (0, i, 0)),
            _whole((1, 128)), _whole((128, 128)), _whole((1, 128)),
        ],
        out_specs=pl.BlockSpec((BR, 128), lambda i: (i, 0)),
        out_shape=jax.ShapeDtypeStruct((N, 128), _f32),
    )(accp, b, wfc, bfc)


@functools.lru_cache(maxsize=None)
def _sc_edge(dout):
    dc = dout + 16
    cg = dout // 16
    mesh = plsc.VectorSubcoreMesh(core_axis_name="c", subcore_axis_name="s",
                                  num_cores=NC, num_subcores=NS)

    @functools.partial(
        pl.kernel,
        out_type=jax.ShapeDtypeStruct((NC, NPAD, dc), _f32),
        mesh=mesh,
        compiler_params=pltpu.CompilerParams(needs_layout_passes=False,
                                             use_tc_tiling_on_sc=False),
        scratch_types=[
            pltpu.VMEM((CHUNK,), jnp.int32),
            pltpu.VMEM((2, CHUNK), jnp.int32),
            pltpu.VMEM((2, CHUNK), _f32),
            pltpu.VMEM((CHUNK,), _f32),
            pltpu.VMEM((2, CHUNK, dc), _f32),
            pltpu.VMEM_SHARED((NPAD, dc), _f32),
            pltpu.SemaphoreType.DMA,
            pltpu.SemaphoreType.DMA,
        ],
    )
    def sc_fn(ed_hbm, src_hbm, dst_hbm, h_hbm, out_hbm,
              sidx, didx, edg, wv, land, acc, gsem, ssem):
        cid = lax.axis_index("c")
        sid = lax.axis_index("s")
        wid = cid * NS + sid
        nchunk = EW // CHUNK

        def _zero_row(r, _):
            for g in range(dc // 16):
                land[0, r, pl.ds(g * 16, 16)] = jnp.zeros((16,), _f32)
            return 0
        lax.fori_loop(0, CHUNK, _zero_row, 0)
        for k in range(RT // CHUNK):
            pltpu.sync_copy(land.at[0],
                            acc.at[pl.ds(sid * RT + k * CHUNK, CHUNK)])
        plsc.subcore_barrier()

        def _iter(j, _):
            b = j % 2

            @pl.when(j >= 2)
            def _drain_scatter():
                pltpu.make_async_copy(
                    h_hbm.at[pl.ds(0, CHUNK)], land.at[0], ssem).wait()

            @pl.when(j < nchunk)
            def _issue():
                base = wid * EW + j * CHUNK
                pltpu.sync_copy(src_hbm.at[pl.ds(base, CHUNK)], sidx)
                pltpu.sync_copy(dst_hbm.at[pl.ds(base, CHUNK)], didx.at[b])
                pltpu.async_copy(h_hbm.at[sidx], land.at[b], gsem)
                pltpu.async_copy(ed_hbm.at[didx.at[b]], edg.at[b], gsem)

            @pl.when(j >= 1)
            def _compute():
                bp = 1 - b
                pltpu.make_async_copy(
                    h_hbm.at[pl.ds(0, CHUNK)], land.at[0], gsem).wait()
                pltpu.make_async_copy(
                    ed_hbm.at[pl.ds(0, CHUNK)], edg.at[0], gsem).wait()

                def _wgrp(g, _):
                    lanes = lax.iota(jnp.int32, 16)
                    col_es = jnp.full((16,), dout, jnp.int32)
                    e = (plsc.load_gather(land.at[bp],
                                          [lanes + g * 16, col_es])
                         + edg[bp, pl.ds(g * 16, 16)])
                    wv[pl.ds(g * 16, 16)] = jnp.exp(jnp.maximum(e, 0.2 * e))
                    return 0
                lax.fori_loop(0, CHUNK // 16, _wgrp, 0)

                def _edge(i, _):
                    wbc = plsc.load_gather(
                        wv, [jnp.full((16,), i, jnp.int32)])
                    for g in range(cg):
                        land[bp, i, pl.ds(g * 16, 16)] = (
                            land[bp, i, pl.ds(g * 16, 16)] * wbc)
                    land[bp, i, pl.ds(dout, 16)] = wbc
                    return 0
                lax.fori_loop(0, CHUNK, _edge, 0)

                pltpu.async_copy(land.at[bp], acc.at[didx.at[bp]],
                                 ssem, add=True)
            return 0
        lax.fori_loop(0, nchunk + 1, _iter, 0)
        pltpu.make_async_copy(h_hbm.at[pl.ds(0, CHUNK)], land.at[0],
                              ssem).wait()

        plsc.subcore_barrier()
        pltpu.sync_copy(acc.at[pl.ds(sid * RT, RT)],
                        out_hbm.at[cid, pl.ds(sid * RT, RT)])

    return sc_fn


def _pad_nodes(v3):
    v = v3.reshape(N)
    return jnp.concatenate([v, jnp.zeros((NPAD - N,), _f32)])


def kernel(x, edge_index, W1, as1, ad1, b1, W2, as2, ad2, b2,
           W3, as3, ad3, b3, W4, as4, ad4, b4, Wfc, bfc):
    loops = jnp.arange(N, dtype=jnp.int32)
    src = jnp.concatenate(
        [edge_index[0], loops, jnp.zeros((EP - NE,), jnp.int32)])
    dst = jnp.concatenate(
        [edge_index[1], loops,
         jnp.full((EP - NE,), PAD_DST, jnp.int32)])

    r2 = lambda a: a.reshape(1, -1)
    h, ed3 = _tc_first(x, W1, r2(as1), r2(ad1))
    accp = _sc_edge(W1.shape[1])(_pad_nodes(ed3), src, dst, h)
    for (W, a_s, a_d, bprev) in ((W2, as2, ad2, b1), (W3, as3, ad3, b2),
                                 (W4, as4, ad4, b3)):
        h, ed3 = _tc_mid(accp, r2(bprev), W, r2(a_s), r2(a_d))
        accp = _sc_edge(W.shape[1])(_pad_nodes(ed3), src, dst, h)
    return _tc_final(accp, r2(b4), Wfc, r2(bfc))
